# baseline (speedup 1.0000x reference)
import functools

import jax
import jax.numpy as jnp
from jax import lax
from jax.experimental import pallas as pl
from jax.experimental.pallas import tpu as pltpu

_SMOOTH = 0.1
_GK = 16


_NS = 4


def _loss_body(*refs, nj, cb, c, rb, nrows):
    x_refs = refs[:_NS]
    out_ref, m_ref, s_ref, sum_ref = refs[_NS:]
    i = pl.program_id(0)
    j = pl.program_id(1)

    @pl.when(j == 0)
    def _init_row_state():
        m_ref[...] = jnp.full((rb, 1), -jnp.inf, dtype=jnp.float32)
        s_ref[...] = jnp.zeros((rb, 1), dtype=jnp.float32)
        sum_ref[...] = jnp.zeros((rb, 1), dtype=jnp.float32)

    @pl.when((i == 0) & (j == 0))
    def _init_out():
        out_ref[0, 0] = 0.0

    col = lax.broadcasted_iota(jnp.int32, (rb, cb), 1)
    for k in range(_NS):
        x = x_refs[k][...]
        limit = c - (j * _NS + k) * cb
        valid = col < limit
        xv = jnp.where(valid, x, -jnp.inf)
        xs = jnp.where(valid, x, 0.0)
        sum_ref[...] += jnp.sum(xs, axis=1, keepdims=True)
        m_old = m_ref[...]
        m_new = jnp.maximum(m_old, jnp.max(xv, axis=1, keepdims=True))
        e = jnp.exp(xv - m_new)
        s_ref[...] = (s_ref[...] * jnp.exp(m_old - m_new)
                      + jnp.sum(e, axis=1, keepdims=True))
        m_ref[...] = m_new

    @pl.when(j == nj - 1)
    def _finalize():
        a = _SMOOTH / (c - 1)
        b = (1.0 - _SMOOTH) - a
        lse = m_ref[...] + jnp.log(s_ref[...])
        row_loss = a * (c * lse - sum_ref[...]) + b * lse
        out_ref[0, 0] += jnp.sum(row_loss) / nrows


def _tc_loss(pred):
    nrows, c = pred.shape
    rb = 256 if nrows % 256 == 0 else nrows
    cb = 2048
    ni = nrows // rb
    njb = (c + cb - 1) // cb
    nj = (njb + _NS - 1) // _NS

    def _mk(k):
        return lambda i, j: (i, jnp.minimum(j * _NS + k, njb - 1))

    out = pl.pallas_call(
        functools.partial(_loss_body, nj=nj, cb=cb, c=c, rb=rb, nrows=nrows),
        grid=(ni, nj),
        in_specs=[pl.BlockSpec((rb, cb), _mk(k)) for k in range(_NS)],
        out_specs=pl.BlockSpec(memory_space=pltpu.SMEM),
        out_shape=jax.ShapeDtypeStruct((1, 1), jnp.float32),
        scratch_shapes=[
            pltpu.VMEM((rb, 1), jnp.float32),
            pltpu.VMEM((rb, 1), jnp.float32),
            pltpu.VMEM((rb, 1), jnp.float32),
        ],
        compiler_params=pltpu.CompilerParams(
            dimension_semantics=("arbitrary", "arbitrary"),
        ),
    )(*([pred] * _NS))
    return out.reshape(())


def _gather_body(t_smem, *refs):
    xs, out_ref = refs[:-1], refs[-1]
    g = pl.program_id(0)

    @pl.when(g == 0)
    def _init():
        out_ref[0, 0] = 0.0

    lane = lax.broadcasted_iota(jnp.int32, (1, 128), 1)
    acc = jnp.zeros((1, 128), jnp.float32)
    for k in range(_GK):
        t_lane = t_smem[g * _GK + k] % 128
        row = xs[k][k % 8:k % 8 + 1, :]
        acc = acc + jnp.where(lane == t_lane, row, 0.0)
    out_ref[0, 0] += jnp.sum(acc)


def _target_sum(pred, target):
    nrows, _ = pred.shape
    grid = nrows // _GK

    def _mk_index_map(k):
        return lambda g, t: (g * (_GK // 8) + k // 8, t[g * _GK + k] // 128)

    out = pl.pallas_call(
        _gather_body,
        grid_spec=pltpu.PrefetchScalarGridSpec(
            num_scalar_prefetch=1,
            grid=(grid,),
            in_specs=[pl.BlockSpec((8, 128), _mk_index_map(k))
                      for k in range(_GK)],
            out_specs=pl.BlockSpec(memory_space=pltpu.SMEM),
        ),
        out_shape=jax.ShapeDtypeStruct((1, 1), jnp.float32),
        compiler_params=pltpu.CompilerParams(
            dimension_semantics=("arbitrary",),
        ),
    )(target.astype(jnp.int32), *([pred] * _GK))
    return out.reshape(())


def kernel(pred, target):
    nrows, c = pred.shape
    tc_part = _tc_loss(pred)
    p_sum = _target_sum(pred, target)
    a = _SMOOTH / (c - 1)
    b = (1.0 - _SMOOTH) - a
    return (tc_part - b * p_sum / nrows).reshape(())


# speedup vs baseline: 1.0271x; 1.0271x over previous
import functools

import jax
import jax.numpy as jnp
from jax import lax
from jax.experimental import pallas as pl
from jax.experimental.pallas import tpu as pltpu

_SMOOTH = 0.1
_GK = 16


_NS = 1
_RB = 512
_CB = 8192


def _loss_body(*refs, nj, cb, c, rb, nrows):
    x_refs = refs[:_NS]
    out_ref, m_ref, s_ref, sum_ref = refs[_NS:]
    i = pl.program_id(0)
    j = pl.program_id(1)

    @pl.when(j == 0)
    def _init_row_state():
        m_ref[...] = jnp.full((rb, 1), -jnp.inf, dtype=jnp.float32)
        s_ref[...] = jnp.zeros((rb, 1), dtype=jnp.float32)
        sum_ref[...] = jnp.zeros((rb, 1), dtype=jnp.float32)

    @pl.when((i == 0) & (j == 0))
    def _init_out():
        out_ref[0, 0] = 0.0

    col = lax.broadcasted_iota(jnp.int32, (rb, cb), 1)
    for k in range(_NS):
        x = x_refs[k][...]
        limit = c - (j * _NS + k) * cb
        valid = col < limit
        xv = jnp.where(valid, x, -jnp.inf)
        xs = jnp.where(valid, x, 0.0)
        sum_ref[...] += jnp.sum(xs, axis=1, keepdims=True)
        m_old = m_ref[...]
        m_new = jnp.maximum(m_old, jnp.max(xv, axis=1, keepdims=True))
        e = jnp.exp(xv - m_new)
        s_ref[...] = (s_ref[...] * jnp.exp(m_old - m_new)
                      + jnp.sum(e, axis=1, keepdims=True))
        m_ref[...] = m_new

    @pl.when(j == nj - 1)
    def _finalize():
        a = _SMOOTH / (c - 1)
        b = (1.0 - _SMOOTH) - a
        lse = m_ref[...] + jnp.log(s_ref[...])
        row_loss = a * (c * lse - sum_ref[...]) + b * lse
        out_ref[0, 0] += jnp.sum(row_loss) / nrows


def _tc_loss(pred):
    nrows, c = pred.shape
    rb = _RB if nrows % _RB == 0 else nrows
    cb = _CB
    ni = nrows // rb
    njb = (c + cb - 1) // cb
    nj = (njb + _NS - 1) // _NS

    def _mk(k):
        return lambda i, j: (i, jnp.minimum(j * _NS + k, njb - 1))

    out = pl.pallas_call(
        functools.partial(_loss_body, nj=nj, cb=cb, c=c, rb=rb, nrows=nrows),
        grid=(ni, nj),
        in_specs=[pl.BlockSpec((rb, cb), _mk(k)) for k in range(_NS)],
        out_specs=pl.BlockSpec(memory_space=pltpu.SMEM),
        out_shape=jax.ShapeDtypeStruct((1, 1), jnp.float32),
        scratch_shapes=[
            pltpu.VMEM((rb, 1), jnp.float32),
            pltpu.VMEM((rb, 1), jnp.float32),
            pltpu.VMEM((rb, 1), jnp.float32),
        ],
        compiler_params=pltpu.CompilerParams(
            dimension_semantics=("arbitrary", "arbitrary"),
        ),
    )(*([pred] * _NS))
    return out.reshape(())


def _gather_body(t_smem, *refs):
    xs, out_ref = refs[:-1], refs[-1]
    g = pl.program_id(0)

    @pl.when(g == 0)
    def _init():
        out_ref[0, 0] = 0.0

    lane = lax.broadcasted_iota(jnp.int32, (1, 128), 1)
    acc = jnp.zeros((1, 128), jnp.float32)
    for k in range(_GK):
        t_lane = t_smem[g * _GK + k] % 128
        row = xs[k][k % 8:k % 8 + 1, :]
        acc = acc + jnp.where(lane == t_lane, row, 0.0)
    out_ref[0, 0] += jnp.sum(acc)


def _target_sum(pred, target):
    nrows, _ = pred.shape
    grid = nrows // _GK

    def _mk_index_map(k):
        return lambda g, t: (g * (_GK // 8) + k // 8, t[g * _GK + k] // 128)

    out = pl.pallas_call(
        _gather_body,
        grid_spec=pltpu.PrefetchScalarGridSpec(
            num_scalar_prefetch=1,
            grid=(grid,),
            in_specs=[pl.BlockSpec((8, 128), _mk_index_map(k))
                      for k in range(_GK)],
            out_specs=pl.BlockSpec(memory_space=pltpu.SMEM),
        ),
        out_shape=jax.ShapeDtypeStruct((1, 1), jnp.float32),
        compiler_params=pltpu.CompilerParams(
            dimension_semantics=("arbitrary",),
        ),
    )(target.astype(jnp.int32), *([pred] * _GK))
    return out.reshape(())


def kernel(pred, target):
    nrows, c = pred.shape
    tc_part = _tc_loss(pred)
    p_sum = _target_sum(pred, target)
    a = _SMOOTH / (c - 1)
    b = (1.0 - _SMOOTH) - a
    return (tc_part - b * p_sum / nrows).reshape(())


# speedup vs baseline: 1.0718x; 1.0435x over previous
import functools

import jax
import jax.numpy as jnp
from jax import lax
from jax.experimental import pallas as pl
from jax.experimental.pallas import tpu as pltpu

_SMOOTH = 0.1
_RB = 32
_NBUF = 4


def _loss_body(t_ref, x_hbm, out_ref, buf, sems, *, c, rb, nrows, nsteps):
    g = pl.program_id(0)

    def _issue(blk, slot):
        pltpu.make_async_copy(
            x_hbm.at[pl.ds(blk * rb, rb), :], buf.at[slot], sems.at[slot]
        ).start()

    @pl.when(g == 0)
    def _warmup():
        out_ref[0, 0] = 0.0
        for b in range(min(_NBUF, nsteps)):
            _issue(b, b)

    slot = lax.rem(g, _NBUF)
    pltpu.make_async_copy(
        x_hbm.at[pl.ds(g * rb, rb), :], buf.at[slot], sems.at[slot]
    ).wait()

    x = buf[slot]
    t_col = t_ref[0]
    col = lax.broadcasted_iota(jnp.int32, (rb, c), 1)
    p = jnp.sum(jnp.where(col == t_col, x, 0.0), axis=1, keepdims=True)
    s_tot = jnp.sum(x, axis=1, keepdims=True)
    m = jnp.max(x, axis=1, keepdims=True)
    e = jnp.exp(x - m)
    lse = m + jnp.log(jnp.sum(e, axis=1, keepdims=True))

    a = _SMOOTH / (c - 1)
    b = (1.0 - _SMOOTH) - a
    row_loss = a * (c * lse - s_tot) + b * (lse - p)
    out_ref[0, 0] += jnp.sum(row_loss) / nrows

    @pl.when(g + _NBUF < nsteps)
    def _refill():
        _issue(g + _NBUF, slot)


def kernel(pred, target):
    nrows, c = pred.shape
    rb = _RB if nrows % _RB == 0 else nrows
    nsteps = nrows // rb

    t3 = target.astype(jnp.int32).reshape(nsteps, rb, 1)

    out = pl.pallas_call(
        functools.partial(_loss_body, c=c, rb=rb, nrows=nrows, nsteps=nsteps),
        grid=(nsteps,),
        in_specs=[
            pl.BlockSpec((1, rb, 1), lambda g: (g, 0, 0)),
            pl.BlockSpec(memory_space=pltpu.MemorySpace.HBM),
        ],
        out_specs=pl.BlockSpec(memory_space=pltpu.SMEM),
        out_shape=jax.ShapeDtypeStruct((1, 1), jnp.float32),
        scratch_shapes=[
            pltpu.VMEM((_NBUF, rb, c), jnp.float32),
            pltpu.SemaphoreType.DMA((_NBUF,)),
        ],
        compiler_params=pltpu.CompilerParams(
            dimension_semantics=("arbitrary",),
        ),
    )(t3, pred)
    return out.reshape(())


# speedup vs baseline: 1.0830x; 1.0105x over previous
import functools

import jax
import jax.numpy as jnp
from jax import lax
from jax.experimental import pallas as pl
from jax.experimental.pallas import tpu as pltpu

_SMOOTH = 0.1
_RB = 32
_NBUF = 4


def _loss_body(t_ref, x_hbm, out_ref, buf, sems, *, c, rb, nrows, nblocks):
    g = pl.program_id(0)

    def _issue(blk, slot):
        pltpu.make_async_copy(
            x_hbm.at[pl.ds(blk * rb, rb), :], buf.at[slot], sems.at[slot]
        ).start()

    @pl.when(g == 0)
    def _warmup():
        out_ref[0, 0] = 0.0
        for b in range(min(_NBUF, nblocks)):
            _issue(b, b)

    a = _SMOOTH / (c - 1)
    bw = (1.0 - _SMOOTH) - a
    col = lax.broadcasted_iota(jnp.int32, (rb, c), 1)

    for b in range(min(_NBUF, nblocks)):
        blk = g * min(_NBUF, nblocks) + b
        pltpu.make_async_copy(
            x_hbm.at[pl.ds(blk * rb, rb), :], buf.at[b], sems.at[b]
        ).wait()

        x = buf[b]
        t_col = t_ref[0, b * rb:(b + 1) * rb, :]
        p = jnp.sum(jnp.where(col == t_col, x, 0.0), axis=1, keepdims=True)
        s_tot = jnp.sum(x, axis=1, keepdims=True)
        m = jnp.max(x, axis=1, keepdims=True)
        e = jnp.exp(x - m)
        lse = m + jnp.log(jnp.sum(e, axis=1, keepdims=True))

        row_loss = a * (c * lse - s_tot) + bw * (lse - p)
        out_ref[0, 0] += jnp.sum(row_loss) / nrows

        @pl.when(blk + _NBUF < nblocks)
        def _refill():
            _issue(blk + _NBUF, b)


def kernel(pred, target):
    nrows, c = pred.shape
    rpg = _RB * _NBUF
    rb = _RB if nrows % rpg == 0 else nrows
    nblocks = nrows // rb
    nsteps = nblocks // _NBUF if nrows % rpg == 0 else 1
    if nrows % rpg != 0:
        nblocks, nsteps = 1, 1

    t3 = target.astype(jnp.int32).reshape(nsteps, nrows // nsteps, 1)

    out = pl.pallas_call(
        functools.partial(_loss_body, c=c, rb=rb, nrows=nrows,
                          nblocks=nblocks),
        grid=(nsteps,),
        in_specs=[
            pl.BlockSpec((1, nrows // nsteps, 1), lambda g: (g, 0, 0)),
            pl.BlockSpec(memory_space=pltpu.MemorySpace.HBM),
        ],
        out_specs=pl.BlockSpec(memory_space=pltpu.SMEM),
        out_shape=jax.ShapeDtypeStruct((1, 1), jnp.float32),
        scratch_shapes=[
            pltpu.VMEM((_NBUF, rb, c), jnp.float32),
            pltpu.SemaphoreType.DMA((_NBUF,)),
        ],
        compiler_params=pltpu.CompilerParams(
            dimension_semantics=("arbitrary",),
        ),
    )(t3, pred)
    return out.reshape(())
